# baseline (speedup 1.0000x reference)
import jax
import jax.numpy as jnp
from jax.experimental import pallas as pl

TOP_K = 2
NUM_EXPERTS = 64
HIDDEN = 2048
TOKENS = 8192

TILE_T = 256
N_STREAMS = 2


def _top2_scores(logits):
    e_iota = jax.lax.broadcasted_iota(jnp.int32, logits.shape, 1).astype(jnp.float32)
    big = jnp.float32(NUM_EXPERTS)

    m1 = jnp.max(logits, axis=1, keepdims=True)
    i1 = jnp.min(jnp.where(logits == m1, e_iota, big), axis=1, keepdims=True)

    masked = jnp.where(e_iota == i1, -jnp.inf, logits)
    m2 = jnp.max(masked, axis=1, keepdims=True)
    i2 = jnp.min(jnp.where(masked == m2, e_iota, big), axis=1, keepdims=True)

    d = jnp.exp(m2 - m1)
    denom = 1.0 + d
    p1 = 1.0 / denom
    p2 = d / denom

    scores = jnp.where(e_iota == i1, p1, jnp.where(e_iota == i2, p2, 0.0))
    idx = jnp.concatenate([i1, i2], axis=1).astype(jnp.int32)
    return scores, idx


def _router_kernel(*refs):
    x_refs = refs[:N_STREAMS]
    wt_ref, b_ref, scores_ref, idx_ref = refs[N_STREAMS:]
    wt = wt_ref[...]
    bias = b_ref[...]
    for k, x_ref in enumerate(x_refs):
        logits = jnp.dot(x_ref[...], wt, preferred_element_type=jnp.float32) + bias
        scores, idx = _top2_scores(logits)
        scores_ref[k * TILE_T:(k + 1) * TILE_T, :] = scores
        idx_ref[k * TILE_T:(k + 1) * TILE_T, :] = idx


def _x_spec(k):
    return pl.BlockSpec((TILE_T, HIDDEN), lambda i, k=k: (N_STREAMS * i + k, 0))


@jax.jit
def kernel(hidden_states, W, b):
    x = hidden_states.reshape(-1, HIDDEN)
    wt = W.T
    b2 = b.reshape(1, NUM_EXPERTS)
    step_t = TILE_T * N_STREAMS
    grid = (TOKENS // step_t,)
    scores, idx = pl.pallas_call(
        _router_kernel,
        grid=grid,
        in_specs=[_x_spec(k) for k in range(N_STREAMS)] + [
            pl.BlockSpec((HIDDEN, NUM_EXPERTS), lambda i: (0, 0)),
            pl.BlockSpec((1, NUM_EXPERTS), lambda i: (0, 0)),
        ],
        out_specs=[
            pl.BlockSpec((step_t, NUM_EXPERTS), lambda i: (i, 0)),
            pl.BlockSpec((step_t, TOP_K), lambda i: (i, 0)),
        ],
        out_shape=[
            jax.ShapeDtypeStruct((TOKENS, NUM_EXPERTS), jnp.float32),
            jax.ShapeDtypeStruct((TOKENS, TOP_K), jnp.int32),
        ],
    )(*([x] * N_STREAMS), wt, b2)
    return scores, idx


# speedup vs baseline: 1.1221x; 1.1221x over previous
import jax
import jax.numpy as jnp
from jax.experimental import pallas as pl

TOP_K = 2
NUM_EXPERTS = 64
HIDDEN = 2048
TOKENS = 8192

TILE_T = 1024
N_STREAMS = 1


def _top2_scores(logits):
    e_iota = jax.lax.broadcasted_iota(jnp.int32, logits.shape, 1).astype(jnp.float32)
    big = jnp.float32(NUM_EXPERTS)

    m1 = jnp.max(logits, axis=1, keepdims=True)
    i1 = jnp.min(jnp.where(logits == m1, e_iota, big), axis=1, keepdims=True)

    masked = jnp.where(e_iota == i1, -jnp.inf, logits)
    m2 = jnp.max(masked, axis=1, keepdims=True)
    i2 = jnp.min(jnp.where(masked == m2, e_iota, big), axis=1, keepdims=True)

    d = jnp.exp(m2 - m1)
    denom = 1.0 + d
    p1 = 1.0 / denom
    p2 = d / denom

    scores = jnp.where(e_iota == i1, p1, jnp.where(e_iota == i2, p2, 0.0))
    idx = jnp.concatenate([i1, i2], axis=1).astype(jnp.int32)
    return scores, idx


def _router_kernel(*refs):
    x_refs = refs[:N_STREAMS]
    wt_ref, b_ref, scores_ref, idx_ref = refs[N_STREAMS:]
    wt = wt_ref[...]
    bias = b_ref[...]
    for k, x_ref in enumerate(x_refs):
        logits = jnp.dot(x_ref[...], wt, preferred_element_type=jnp.float32) + bias
        scores, idx = _top2_scores(logits)
        scores_ref[k * TILE_T:(k + 1) * TILE_T, :] = scores
        idx_ref[k * TILE_T:(k + 1) * TILE_T, :] = idx


def _x_spec(k):
    return pl.BlockSpec((TILE_T, HIDDEN), lambda i, k=k: (N_STREAMS * i + k, 0))


@jax.jit
def kernel(hidden_states, W, b):
    x = hidden_states.reshape(-1, HIDDEN)
    wt = W.T
    b2 = b.reshape(1, NUM_EXPERTS)
    step_t = TILE_T * N_STREAMS
    grid = (TOKENS // step_t,)
    scores, idx = pl.pallas_call(
        _router_kernel,
        grid=grid,
        in_specs=[_x_spec(k) for k in range(N_STREAMS)] + [
            pl.BlockSpec((HIDDEN, NUM_EXPERTS), lambda i: (0, 0)),
            pl.BlockSpec((1, NUM_EXPERTS), lambda i: (0, 0)),
        ],
        out_specs=[
            pl.BlockSpec((step_t, NUM_EXPERTS), lambda i: (i, 0)),
            pl.BlockSpec((step_t, TOP_K), lambda i: (i, 0)),
        ],
        out_shape=[
            jax.ShapeDtypeStruct((TOKENS, NUM_EXPERTS), jnp.float32),
            jax.ShapeDtypeStruct((TOKENS, TOP_K), jnp.int32),
        ],
    )(*([x] * N_STREAMS), wt, b2)
    return scores, idx
